# baseline (speedup 1.0000x reference)
import jax
import jax.numpy as jnp
import numpy as np
from jax.experimental import pallas as pl
from jax.experimental.pallas import tpu as pltpu

_B, _S, _D_IN, _D_OUT = 2, 2048, 2048, 2048
_E, _R = 8, 64
_RMOE = _E * _R
_KC = _D_IN + _RMOE
_SCALING = 16.0 / 64.0

_BM = 256


def _body(x_ref, wb_ref, wg_ref, wa_ref, wbl_ref, o_ref,
          wcomb_ref, wa16_ref, xcomb_ref):
    @pl.when(pl.program_id(0) == 0)
    def _cast_weights():
        wcomb_ref[:, :_D_IN] = wb_ref[...].astype(jnp.bfloat16)
        wcomb_ref[:, _D_IN:] = (_SCALING * wbl_ref[...]).astype(jnp.bfloat16)
        wa16_ref[...] = wa_ref[...].astype(jnp.bfloat16)

    xb = x_ref[...].astype(jnp.bfloat16)
    xcomb_ref[:, :_D_IN] = xb

    lT = jax.lax.dot_general(
        wg_ref[...].astype(jnp.bfloat16), xb, (((1,), (1,)), ((), ())),
        preferred_element_type=jnp.float32)
    lj = lT[:, None, :]
    le = lT[None, :, :]
    j_idx = jax.lax.broadcasted_iota(jnp.int32, (_E, _E, _BM), 0)
    e_idx = jax.lax.broadcasted_iota(jnp.int32, (_E, _E, _BM), 1)
    beats = (lj > le) | ((lj == le) & (j_idx < e_idx))
    rank = jnp.sum(beats.astype(jnp.int32), axis=0)
    m1 = jnp.max(lT, axis=0, keepdims=True)
    wun = jnp.where(rank < 2, jnp.exp(lT - m1), 0.0)
    wtsT = wun / jnp.sum(wun, axis=0, keepdims=True)

    expand = (jax.lax.broadcasted_iota(jnp.int32, (_E, _RMOE), 1) // _R ==
              jax.lax.broadcasted_iota(jnp.int32, (_E, _RMOE), 0)
              ).astype(jnp.float32)
    wfull = jax.lax.dot_general(
        wtsT, expand, (((0,), (0,)), ((), ())),
        preferred_element_type=jnp.float32)

    h = jax.lax.dot_general(
        xb, wa16_ref[...], (((1,), (1,)), ((), ())),
        preferred_element_type=jnp.float32)
    xcomb_ref[:, _D_IN:] = (h * wfull).astype(jnp.bfloat16)

    o_ref[...] = jax.lax.dot_general(
        xcomb_ref[...], wcomb_ref[...], (((1,), (1,)), ((), ())),
        preferred_element_type=jnp.float32)


def kernel(x, W_base, W_gate, W_A, W_B):
    xf = x.reshape(_B * _S, _D_IN)

    n_blocks = (_B * _S) // _BM
    out = pl.pallas_call(
        _body,
        grid=(n_blocks,),
        in_specs=[
            pl.BlockSpec((_BM, _D_IN), lambda i: (i, 0)),
            pl.BlockSpec((_D_OUT, _D_IN), lambda i: (0, 0)),
            pl.BlockSpec((_E, _D_IN), lambda i: (0, 0)),
            pl.BlockSpec((_RMOE, _D_IN), lambda i: (0, 0)),
            pl.BlockSpec((_D_OUT, _RMOE), lambda i: (0, 0)),
        ],
        out_specs=pl.BlockSpec((_BM, _D_OUT), lambda i: (i, 0)),
        out_shape=jax.ShapeDtypeStruct((_B * _S, _D_OUT), jnp.float32),
        scratch_shapes=[
            pltpu.VMEM((_D_OUT, _KC), jnp.bfloat16),
            pltpu.VMEM((_RMOE, _D_IN), jnp.bfloat16),
            pltpu.VMEM((_BM, _KC), jnp.bfloat16),
        ],
        compiler_params=pltpu.CompilerParams(
            dimension_semantics=("arbitrary",),
            vmem_limit_bytes=100 * 1024 * 1024,
        ),
    )(xf, W_base, W_gate, W_A, W_B)
    return out.reshape(_B, _S, _D_OUT)


# speedup vs baseline: 1.0538x; 1.0538x over previous
import jax
import jax.numpy as jnp
import numpy as np
from jax.experimental import pallas as pl
from jax.experimental.pallas import tpu as pltpu

_B, _S, _D_IN, _D_OUT = 2, 2048, 2048, 2048
_E, _R = 8, 64
_RMOE = _E * _R
_KC = _D_IN + _RMOE
_SCALING = 16.0 / 64.0

_BM = 512
_NCHUNK = 4
_CH = _D_OUT // _NCHUNK


def _body(x_ref, wb_ref, wg_ref, wa_ref, wbl_ref, o_ref,
          wcomb_ref, wa16_ref, xcomb_ref, wstage_ref, wblstage_ref, sems):
    @pl.when(pl.program_id(0) == 0)
    def _cast_weights():
        for c in range(_NCHUNK):
            pltpu.make_async_copy(
                wb_ref.at[pl.ds(c * _CH, _CH), :], wstage_ref.at[c],
                sems.at[c]).start()
        pltpu.make_async_copy(wbl_ref, wblstage_ref, sems.at[_NCHUNK]).start()
        wa16_ref[...] = wa_ref[...].astype(jnp.bfloat16)
        for c in range(_NCHUNK):
            pltpu.make_async_copy(
                wb_ref.at[pl.ds(c * _CH, _CH), :], wstage_ref.at[c],
                sems.at[c]).wait()
            wcomb_ref[pl.ds(c * _CH, _CH), :_D_IN] = (
                wstage_ref[c].astype(jnp.bfloat16))
        pltpu.make_async_copy(wbl_ref, wblstage_ref, sems.at[_NCHUNK]).wait()
        wcomb_ref[:, _D_IN:] = (_SCALING * wblstage_ref[...]).astype(jnp.bfloat16)

    xb = x_ref[...].astype(jnp.bfloat16)
    xcomb_ref[:, :_D_IN] = xb

    lT = jax.lax.dot_general(
        wg_ref[...].astype(jnp.bfloat16), xb, (((1,), (1,)), ((), ())),
        preferred_element_type=jnp.float32)
    lj = lT[:, None, :]
    le = lT[None, :, :]
    j_idx = jax.lax.broadcasted_iota(jnp.int32, (_E, _E, _BM), 0)
    e_idx = jax.lax.broadcasted_iota(jnp.int32, (_E, _E, _BM), 1)
    beats = (lj > le) | ((lj == le) & (j_idx < e_idx))
    rank = jnp.sum(beats.astype(jnp.int32), axis=0)
    m1 = jnp.max(lT, axis=0, keepdims=True)
    wun = jnp.where(rank < 2, jnp.exp(lT - m1), 0.0)
    wtsT = wun / jnp.sum(wun, axis=0, keepdims=True)

    expand = (jax.lax.broadcasted_iota(jnp.int32, (_E, _RMOE), 1) // _R ==
              jax.lax.broadcasted_iota(jnp.int32, (_E, _RMOE), 0)
              ).astype(jnp.float32)
    wfull = jax.lax.dot_general(
        wtsT, expand, (((0,), (0,)), ((), ())),
        preferred_element_type=jnp.float32)

    h = jax.lax.dot_general(
        xb, wa16_ref[...], (((1,), (1,)), ((), ())),
        preferred_element_type=jnp.float32)
    xcomb_ref[:, _D_IN:] = (h * wfull).astype(jnp.bfloat16)

    o_ref[...] = jax.lax.dot_general(
        xcomb_ref[...], wcomb_ref[...], (((1,), (1,)), ((), ())),
        preferred_element_type=jnp.float32)


def kernel(x, W_base, W_gate, W_A, W_B):
    xf = x.reshape(_B * _S, _D_IN)

    n_blocks = (_B * _S) // _BM
    out = pl.pallas_call(
        _body,
        grid=(n_blocks,),
        in_specs=[
            pl.BlockSpec((_BM, _D_IN), lambda i: (i, 0)),
            pl.BlockSpec(memory_space=pltpu.MemorySpace.HBM),
            pl.BlockSpec((_E, _D_IN), lambda i: (0, 0)),
            pl.BlockSpec((_RMOE, _D_IN), lambda i: (0, 0)),
            pl.BlockSpec(memory_space=pltpu.MemorySpace.HBM),
        ],
        out_specs=pl.BlockSpec((_BM, _D_OUT), lambda i: (i, 0)),
        out_shape=jax.ShapeDtypeStruct((_B * _S, _D_OUT), jnp.float32),
        scratch_shapes=[
            pltpu.VMEM((_D_OUT, _KC), jnp.bfloat16),
            pltpu.VMEM((_RMOE, _D_IN), jnp.bfloat16),
            pltpu.VMEM((_BM, _KC), jnp.bfloat16),
            pltpu.VMEM((_NCHUNK, _CH, _D_IN), jnp.float32),
            pltpu.VMEM((_D_OUT, _RMOE), jnp.float32),
            pltpu.SemaphoreType.DMA((_NCHUNK + 1,)),
        ],
        compiler_params=pltpu.CompilerParams(
            dimension_semantics=("arbitrary",),
            vmem_limit_bytes=100 * 1024 * 1024,
        ),
    )(xf, W_base, W_gate, W_A, W_B)
    return out.reshape(_B, _S, _D_OUT)


# speedup vs baseline: 1.0575x; 1.0034x over previous
import jax
import jax.numpy as jnp
from jax.experimental import pallas as pl
from jax.experimental.pallas import tpu as pltpu

_B, _S, _D_IN, _D_OUT = 2, 2048, 2048, 2048
_E, _R = 8, 64
_RMOE = _E * _R
_KC = _D_IN + _RMOE
_SCALING = 16.0 / 64.0

_BM = 512


def _body(x_ref, wb_ref, wg_ref, wa_ref, wbl_ref, o_ref,
          wcomb_ref, wa16_ref, xcomb_ref):
    @pl.when(pl.program_id(0) == 0)
    def _cast_weights():
        wcomb_ref[:, :_D_IN] = wb_ref[...].astype(jnp.bfloat16)
        wcomb_ref[:, _D_IN:] = (_SCALING * wbl_ref[...]).astype(jnp.bfloat16)
        wa16_ref[...] = wa_ref[...].astype(jnp.bfloat16)

    xb = x_ref[...].astype(jnp.bfloat16)
    xcomb_ref[:, :_D_IN] = xb

    lT = jax.lax.dot_general(
        wg_ref[...].astype(jnp.bfloat16), xb, (((1,), (1,)), ((), ())),
        preferred_element_type=jnp.float32)
    lj = lT[:, None, :]
    le = lT[None, :, :]
    j_idx = jax.lax.broadcasted_iota(jnp.int32, (_E, _E, _BM), 0)
    e_idx = jax.lax.broadcasted_iota(jnp.int32, (_E, _E, _BM), 1)
    beats = (lj > le) | ((lj == le) & (j_idx < e_idx))
    rank = jnp.sum(beats.astype(jnp.int32), axis=0)
    m1 = jnp.max(lT, axis=0, keepdims=True)
    wun = jnp.where(rank < 2, jnp.exp(lT - m1), 0.0)
    wtsT = wun / jnp.sum(wun, axis=0, keepdims=True)

    expand = (jax.lax.broadcasted_iota(jnp.int32, (_E, _RMOE), 1) // _R ==
              jax.lax.broadcasted_iota(jnp.int32, (_E, _RMOE), 0)
              ).astype(jnp.float32)
    wfull = jax.lax.dot_general(
        wtsT, expand, (((0,), (0,)), ((), ())),
        preferred_element_type=jnp.float32)

    h = jax.lax.dot_general(
        xb, wa16_ref[...], (((1,), (1,)), ((), ())),
        preferred_element_type=jnp.float32)
    xcomb_ref[:, _D_IN:] = (h * wfull).astype(jnp.bfloat16)

    o_ref[...] = jax.lax.dot_general(
        xcomb_ref[...], wcomb_ref[...], (((1,), (1,)), ((), ())),
        preferred_element_type=jnp.float32)


def kernel(x, W_base, W_gate, W_A, W_B):
    xf = x.reshape(_B * _S, _D_IN)

    n_blocks = (_B * _S) // _BM
    out = pl.pallas_call(
        _body,
        grid=(n_blocks,),
        in_specs=[
            pl.BlockSpec((_BM, _D_IN), lambda i: (i, 0)),
            pl.BlockSpec((_D_OUT, _D_IN), lambda i: (0, 0)),
            pl.BlockSpec((_E, _D_IN), lambda i: (0, 0)),
            pl.BlockSpec((_RMOE, _D_IN), lambda i: (0, 0)),
            pl.BlockSpec((_D_OUT, _RMOE), lambda i: (0, 0)),
        ],
        out_specs=pl.BlockSpec((_BM, _D_OUT), lambda i: (i, 0)),
        out_shape=jax.ShapeDtypeStruct((_B * _S, _D_OUT), jnp.float32),
        scratch_shapes=[
            pltpu.VMEM((_D_OUT, _KC), jnp.bfloat16),
            pltpu.VMEM((_RMOE, _D_IN), jnp.bfloat16),
            pltpu.VMEM((_BM, _KC), jnp.bfloat16),
        ],
        compiler_params=pltpu.CompilerParams(
            dimension_semantics=("arbitrary",),
            vmem_limit_bytes=100 * 1024 * 1024,
        ),
    )(xf, W_base, W_gate, W_A, W_B)
    return out.reshape(_B, _S, _D_OUT)
